# Initial kernel scaffold; baseline (speedup 1.0000x reference)
import jax
import jax.numpy as jnp
from jax.experimental import pallas as pl


def kernel(z, emb_w, proj_w, proj_b):
    raise NotImplementedError("write your pallas kernel here")


# speedup vs baseline: 2.1870x; 2.1870x over previous
import functools

import jax
import jax.numpy as jnp
from jax.experimental import pallas as pl
from jax.experimental.pallas import tpu as pltpu

K_CODES = 8192
DIM = 32
N_TOK = 8192
TB = 256
NB = N_TOK // TB
BETA_C = 0.25
ENT_RATIO = 0.1
TEMPERATURE = 0.01


def _vq_kernel(z_ref, emb_ref, pw_ref, pb_ref,
               q_ref, idx_ref, cb_ref, cm_ref, ent_ref,
               w_s, w2_s, avgp_s, acc_s):
    i = pl.program_id(0)

    @pl.when(i == 0)
    def _init():
        w = jax.lax.dot_general(
            emb_ref[...], pw_ref[...], (((1,), (1,)), ((), ())),
            preferred_element_type=jnp.float32) + pb_ref[...]
        w_s[...] = w
        w2_s[...] = jnp.sum(w * w, axis=1).reshape(1, K_CODES)
        avgp_s[...] = jnp.zeros((1, K_CODES), jnp.float32)
        acc_s[0] = 0.0
        acc_s[1] = 0.0

    z_blk = z_ref[...]
    w = w_s[...]
    dot = jax.lax.dot_general(
        z_blk, w, (((1,), (1,)), ((), ())),
        preferred_element_type=jnp.float32)
    z2 = jnp.sum(z_blk * z_blk, axis=1, keepdims=True)
    d = z2 + w2_s[...] - 2.0 * dot

    dmin = jnp.min(d, axis=1)
    iota = jax.lax.broadcasted_iota(jnp.int32, (TB, K_CODES), 1)
    idx = jnp.min(jnp.where(d == dmin[:, None], iota, K_CODES), axis=1)
    idx_ref[0, 0, :] = idx

    onehot = (iota == idx[:, None]).astype(jnp.float32)
    q_blk = jax.lax.dot_general(
        onehot, w, (((1,), (0,)), ((), ())),
        preferred_element_type=jnp.float32,
        precision=jax.lax.Precision.HIGHEST)
    q_ref[...] = q_blk
    dq = q_blk - z_blk
    acc_s[0] += jnp.sum(dq * dq)

    la = (-d) / TEMPERATURE
    m = (-dmin) / TEMPERATURE
    e = jnp.exp(la - m[:, None])
    zsum = jnp.sum(e, axis=1)
    sl = jnp.sum(e * la, axis=1)
    acc_s[1] += jnp.sum(sl / zsum - m - jnp.log(zsum))
    avgp_s[...] += jnp.sum(e / zsum[:, None], axis=0).reshape(1, K_CODES)

    @pl.when(i == NB - 1)
    def _finalize():
        inv_n = jnp.float32(1.0 / N_TOK)
        cb = acc_s[0] * jnp.float32(1.0 / (N_TOK * DIM))
        cb_ref[...] = jnp.full((1, 1), cb, jnp.float32)
        cm_ref[...] = jnp.full((1, 1), BETA_C * cb, jnp.float32)
        ap = avgp_s[...] * inv_n
        avg_ent = -jnp.sum(ap * jnp.log(ap + 1e-5))
        sample_ent = -(acc_s[1] * inv_n)
        ent_ref[...] = jnp.full((1, 1), ENT_RATIO * (sample_ent - avg_ent),
                                jnp.float32)


@functools.partial(jax.jit, static_argnames=())
def _run(z_flat, emb_w, proj_w, proj_b2):
    out_shapes = (
        jax.ShapeDtypeStruct((N_TOK, DIM), jnp.float32),
        jax.ShapeDtypeStruct((NB, 1, TB), jnp.int32),
        jax.ShapeDtypeStruct((1, 1), jnp.float32),
        jax.ShapeDtypeStruct((1, 1), jnp.float32),
        jax.ShapeDtypeStruct((1, 1), jnp.float32),
    )
    return pl.pallas_call(
        _vq_kernel,
        grid=(NB,),
        in_specs=[
            pl.BlockSpec((TB, DIM), lambda i: (i, 0)),
            pl.BlockSpec((K_CODES, DIM), lambda i: (0, 0)),
            pl.BlockSpec((DIM, DIM), lambda i: (0, 0)),
            pl.BlockSpec((1, DIM), lambda i: (0, 0)),
        ],
        out_specs=(
            pl.BlockSpec((TB, DIM), lambda i: (i, 0)),
            pl.BlockSpec((1, 1, TB), lambda i: (i, 0, 0)),
            pl.BlockSpec((1, 1), lambda i: (0, 0)),
            pl.BlockSpec((1, 1), lambda i: (0, 0)),
            pl.BlockSpec((1, 1), lambda i: (0, 0)),
        ),
        out_shape=out_shapes,
        scratch_shapes=[
            pltpu.VMEM((K_CODES, DIM), jnp.float32),
            pltpu.VMEM((1, K_CODES), jnp.float32),
            pltpu.VMEM((1, K_CODES), jnp.float32),
            pltpu.SMEM((2,), jnp.float32),
        ],
    )(z_flat, emb_w, proj_w, proj_b2)


def kernel(z, emb_w, proj_w, proj_b):
    b, c, h, w = z.shape
    z_bhwc = jnp.transpose(z, (0, 2, 3, 1))
    z_flat = z_bhwc.reshape(N_TOK, DIM)
    q, idx, cb, cm, ent = _run(z_flat, emb_w, proj_w, proj_b.reshape(1, DIM))
    z_q = jnp.transpose(q.reshape(b, h, w, c), (0, 3, 1, 2))
    flat_indices = idx.reshape(N_TOK)
    usage = jnp.float32(0.0)
    return (z_q, cb[0, 0], cm[0, 0], ent[0, 0], usage, flat_indices)


# speedup vs baseline: 4.0193x; 1.8378x over previous
import functools

import jax
import jax.numpy as jnp
from jax.experimental import pallas as pl
from jax.experimental.pallas import tpu as pltpu
from jax.experimental.pallas import tpu_sc as plsc

K_CODES = 8192
DIM = 32
N_TOK = 8192
TB = 256
NB = N_TOK // TB
BETA_C = 0.25
ENT_RATIO = 0.1
TEMPERATURE = 0.01
GW = 128


def _vq_kernel(z_ref, emb_ref, pw_ref, pb_ref,
               wpad_ref, idx_ref, cb_ref, cm_ref, ent_ref,
               w_s, w2_s, avgp_s, acc_s):
    i = pl.program_id(0)

    @pl.when(i == 0)
    def _init():
        w = jax.lax.dot_general(
            emb_ref[...], pw_ref[...], (((1,), (1,)), ((), ())),
            preferred_element_type=jnp.float32) + pb_ref[...]
        w_s[...] = w
        wpad_ref[...] = jnp.concatenate(
            [w, jnp.zeros((K_CODES, 128 - DIM), jnp.float32)], axis=1)
        w2_s[...] = jnp.sum(w * w, axis=1).reshape(1, K_CODES)
        avgp_s[...] = jnp.zeros((1, K_CODES), jnp.float32)
        acc_s[0] = 0.0
        acc_s[1] = 0.0

    z_blk = z_ref[...]
    w = w_s[...]
    dot = jax.lax.dot_general(
        z_blk, w, (((1,), (1,)), ((), ())),
        preferred_element_type=jnp.float32)
    z2 = jnp.sum(z_blk * z_blk, axis=1, keepdims=True)
    d = z2 + w2_s[...] - 2.0 * dot

    dmin = jnp.min(d, axis=1)
    iota = jax.lax.broadcasted_iota(jnp.int32, (TB, K_CODES), 1)
    idx = jnp.min(jnp.where(d == dmin[:, None], iota, K_CODES), axis=1)
    idx_ref[0, 0, :] = idx

    acc_s[0] += jnp.sum(dmin)

    la = (-d) / TEMPERATURE
    m = (-dmin) / TEMPERATURE
    e = jnp.exp(la - m[:, None])
    zsum = jnp.sum(e, axis=1)
    sl = jnp.sum(e * la, axis=1)
    acc_s[1] += jnp.sum(sl / zsum - m - jnp.log(zsum))
    avgp_s[...] += jnp.sum(e / zsum[:, None], axis=0).reshape(1, K_CODES)

    @pl.when(i == NB - 1)
    def _finalize():
        inv_n = jnp.float32(1.0 / N_TOK)
        cb = acc_s[0] * jnp.float32(1.0 / (N_TOK * DIM))
        cb_ref[...] = jnp.full((1, 1), cb, jnp.float32)
        cm_ref[...] = jnp.full((1, 1), BETA_C * cb, jnp.float32)
        ap = avgp_s[...] * inv_n
        avg_ent = -jnp.sum(ap * jnp.log(ap + 1e-5))
        sample_ent = -(acc_s[1] * inv_n)
        ent_ref[...] = jnp.full((1, 1), ENT_RATIO * (sample_ent - avg_ent),
                                jnp.float32)


def _tc_stage(z_flat, emb_w, proj_w, proj_b2):
    out_shapes = (
        jax.ShapeDtypeStruct((K_CODES, 128), jnp.float32),
        jax.ShapeDtypeStruct((NB, 1, TB), jnp.int32),
        jax.ShapeDtypeStruct((1, 1), jnp.float32),
        jax.ShapeDtypeStruct((1, 1), jnp.float32),
        jax.ShapeDtypeStruct((1, 1), jnp.float32),
    )
    return pl.pallas_call(
        _vq_kernel,
        grid=(NB,),
        in_specs=[
            pl.BlockSpec((TB, DIM), lambda i: (i, 0)),
            pl.BlockSpec((K_CODES, DIM), lambda i: (0, 0)),
            pl.BlockSpec((DIM, DIM), lambda i: (0, 0)),
            pl.BlockSpec((1, DIM), lambda i: (0, 0)),
        ],
        out_specs=(
            pl.BlockSpec((K_CODES, 128), lambda i: (0, 0)),
            pl.BlockSpec((1, 1, TB), lambda i: (i, 0, 0)),
            pl.BlockSpec((1, 1), lambda i: (0, 0)),
            pl.BlockSpec((1, 1), lambda i: (0, 0)),
            pl.BlockSpec((1, 1), lambda i: (0, 0)),
        ),
        out_shape=out_shapes,
        scratch_shapes=[
            pltpu.VMEM((K_CODES, DIM), jnp.float32),
            pltpu.VMEM((1, K_CODES), jnp.float32),
            pltpu.VMEM((1, K_CODES), jnp.float32),
            pltpu.SMEM((2,), jnp.float32),
        ],
    )(z_flat, emb_w, proj_w, proj_b2)


def _sc_gather(weight, indices2d):
    mesh = plsc.VectorSubcoreMesh(core_axis_name="core",
                                  subcore_axis_name="subcore")

    @pl.kernel(out_type=jax.ShapeDtypeStruct((N_TOK, 128), jnp.float32),
               mesh=mesh)
    def kern(w_hbm, i_hbm, o_hbm):
        def body(i_vmem, o_vmem):
            pltpu.sync_copy(w_hbm.at[i_vmem.at[0]], o_vmem)

        pltpu.emit_pipeline(
            body,
            grid=(N_TOK // GW,),
            in_specs=[pl.BlockSpec((1, GW), index_map=lambda i: (0, i))],
            out_specs=[pl.BlockSpec((GW, 128), index_map=lambda i: (i, 0))],
            core_axis_name=("core", "subcore"),
            dimension_semantics=(pltpu.PARALLEL,),
        )(i_hbm, o_hbm)

    return kern(weight, indices2d)


@jax.jit
def _run(z_flat, emb_w, proj_w, proj_b2):
    w, idx, cb, cm, ent = _tc_stage(z_flat, emb_w, proj_w, proj_b2)
    q = _sc_gather(w, idx.reshape(1, N_TOK))[:, :DIM]
    return q, idx, cb, cm, ent


def kernel(z, emb_w, proj_w, proj_b):
    b, c, h, w = z.shape
    z_bhwc = jnp.transpose(z, (0, 2, 3, 1))
    z_flat = z_bhwc.reshape(N_TOK, DIM)
    q, idx, cb, cm, ent = _run(z_flat, emb_w, proj_w, proj_b.reshape(1, DIM))
    z_q = jnp.transpose(q.reshape(b, h, w, c), (0, 3, 1, 2))
    flat_indices = idx.reshape(N_TOK)
    usage = jnp.float32(0.0)
    return (z_q, cb[0, 0], cm[0, 0], ent[0, 0], usage, flat_indices)


# speedup vs baseline: 4.9077x; 1.2210x over previous
import jax
import jax.numpy as jnp
from jax.experimental import pallas as pl
from jax.experimental.pallas import tpu as pltpu
from jax.experimental.pallas import tpu_sc as plsc

K_CODES = 8192
DIM = 32
N_TOK = 8192
TB = 256
NB = N_TOK // TB
BETA_C = 0.25
ENT_RATIO = 0.1
INV_TEMP = 100.0
GW = 128


def _proj_kernel(emb_ref, pw_ref, pb_ref, w_ref, wpad_ref, w2_ref):
    w = jax.lax.dot_general(
        emb_ref[...], pw_ref[...], (((1,), (1,)), ((), ())),
        preferred_element_type=jnp.float32) + pb_ref[...]
    w_ref[...] = w
    wpad_ref[...] = jnp.concatenate(
        [w, jnp.zeros((K_CODES, 128 - DIM), jnp.float32)], axis=1)
    w2_ref[...] = jnp.sum(w * w, axis=1).reshape(1, K_CODES)


def _proj_stage(emb_w, proj_w, proj_b2):
    return pl.pallas_call(
        _proj_kernel,
        out_shape=(
            jax.ShapeDtypeStruct((K_CODES, DIM), jnp.float32),
            jax.ShapeDtypeStruct((K_CODES, 128), jnp.float32),
            jax.ShapeDtypeStruct((1, K_CODES), jnp.float32),
        ),
    )(emb_w, proj_w, proj_b2)


def _vq_kernel(z_ref, w_ref, w2_ref,
               idx_ref, cb_ref, cm_ref, ent_ref,
               avgp_s, acc_s):
    i = pl.program_id(0)

    @pl.when(i == 0)
    def _init():
        avgp_s[...] = jnp.zeros((1, K_CODES), jnp.float32)
        acc_s[0] = 0.0
        acc_s[1] = 0.0

    z_blk = z_ref[...]
    dot = jax.lax.dot_general(
        z_blk, w_ref[...], (((1,), (1,)), ((), ())),
        preferred_element_type=jnp.float32)
    z2 = jnp.sum(z_blk * z_blk, axis=1, keepdims=True)
    d = z2 + w2_ref[...] - 2.0 * dot

    dmin = jnp.min(d, axis=1)
    iota = jax.lax.broadcasted_iota(jnp.int32, (TB, K_CODES), 1)
    idx = jnp.min(jnp.where(d == dmin[:, None], iota, K_CODES), axis=1)
    idx_ref[0, 0, :] = idx

    acc_s[0] += jnp.sum(dmin)

    t = (dmin[:, None] - d) * INV_TEMP
    e = jnp.exp(t)
    zsum = jnp.sum(e, axis=1)
    sl = jnp.sum(e * t, axis=1)
    acc_s[1] += jnp.sum(sl / zsum - jnp.log(zsum))
    avgp_s[...] += jnp.sum(e / zsum[:, None], axis=0).reshape(1, K_CODES)

    @pl.when(i == NB - 1)
    def _finalize():
        inv_n = jnp.float32(1.0 / N_TOK)
        cb = acc_s[0] * jnp.float32(1.0 / (N_TOK * DIM))
        cb_ref[...] = jnp.full((1, 1), cb, jnp.float32)
        cm_ref[...] = jnp.full((1, 1), BETA_C * cb, jnp.float32)
        ap = avgp_s[...] * inv_n
        avg_ent = -jnp.sum(ap * jnp.log(ap + 1e-5))
        sample_ent = -(acc_s[1] * inv_n)
        ent_ref[...] = jnp.full((1, 1), ENT_RATIO * (sample_ent - avg_ent),
                                jnp.float32)


def _tc_stage(z_flat, w, w2):
    out_shapes = (
        jax.ShapeDtypeStruct((NB, 1, TB), jnp.int32),
        jax.ShapeDtypeStruct((1, 1), jnp.float32),
        jax.ShapeDtypeStruct((1, 1), jnp.float32),
        jax.ShapeDtypeStruct((1, 1), jnp.float32),
    )
    return pl.pallas_call(
        _vq_kernel,
        grid=(NB,),
        in_specs=[
            pl.BlockSpec((TB, DIM), lambda i: (i, 0)),
            pl.BlockSpec((K_CODES, DIM), lambda i: (0, 0)),
            pl.BlockSpec((1, K_CODES), lambda i: (0, 0)),
        ],
        out_specs=(
            pl.BlockSpec((1, 1, TB), lambda i: (i, 0, 0)),
            pl.BlockSpec((1, 1), lambda i: (0, 0)),
            pl.BlockSpec((1, 1), lambda i: (0, 0)),
            pl.BlockSpec((1, 1), lambda i: (0, 0)),
        ),
        out_shape=out_shapes,
        scratch_shapes=[
            pltpu.VMEM((1, K_CODES), jnp.float32),
            pltpu.SMEM((2,), jnp.float32),
        ],
    )(z_flat, w, w2)


def _sc_gather(weight_pad, indices2d):
    mesh = plsc.VectorSubcoreMesh(core_axis_name="core",
                                  subcore_axis_name="subcore")

    @pl.kernel(out_type=jax.ShapeDtypeStruct((N_TOK, 128), jnp.float32),
               mesh=mesh)
    def kern(w_hbm, i_hbm, o_hbm):
        def body(i_vmem, o_vmem):
            pltpu.sync_copy(w_hbm.at[i_vmem.at[0]], o_vmem)

        pltpu.emit_pipeline(
            body,
            grid=(N_TOK // GW,),
            in_specs=[pl.BlockSpec((1, GW), index_map=lambda i: (0, i))],
            out_specs=[pl.BlockSpec((GW, 128), index_map=lambda i: (i, 0))],
            core_axis_name=("core", "subcore"),
            dimension_semantics=(pltpu.PARALLEL,),
        )(i_hbm, o_hbm)

    return kern(weight_pad, indices2d)


@jax.jit
def _run(z_flat, emb_w, proj_w, proj_b2):
    w, wpad, w2 = _proj_stage(emb_w, proj_w, proj_b2)
    idx, cb, cm, ent = _tc_stage(z_flat, w, w2)
    q = _sc_gather(wpad, idx.reshape(1, N_TOK))[:, :DIM]
    return q, idx, cb, cm, ent


def kernel(z, emb_w, proj_w, proj_b):
    b, c, h, w = z.shape
    z_bhwc = jnp.transpose(z, (0, 2, 3, 1))
    z_flat = z_bhwc.reshape(N_TOK, DIM)
    q, idx, cb, cm, ent = _run(z_flat, emb_w, proj_w, proj_b.reshape(1, DIM))
    z_q = jnp.transpose(q.reshape(b, h, w, c), (0, 3, 1, 2))
    flat_indices = idx.reshape(N_TOK)
    usage = jnp.float32(0.0)
    return (z_q, cb[0, 0], cm[0, 0], ent[0, 0], usage, flat_indices)
